# Initial kernel scaffold; baseline (speedup 1.0000x reference)
import jax
import jax.numpy as jnp
from jax import lax
from jax.experimental import pallas as pl
from jax.experimental.pallas import tpu as pltpu
from jax.experimental.pallas import tpu_sc as plsc

N_NODES = 10000
E_EDGES = 320000
D = 128
NC, NS = 2, 16
NW = NC * NS
E_PER_W = E_EDGES // NW
N_PAD = 10240
ROWS_PER_SUB = N_PAD // NS
DEG_K = 2000
MSG_K = 400
ZR = 64

_mesh = plsc.VectorSubcoreMesh(core_axis_name="c", subcore_axis_name="s")


def _zero_shared(zrows, acc_sh, width, sub):
    zv = jnp.zeros((16,), jnp.float32)

    @pl.loop(0, ZR)
    def _(i):
        @pl.loop(0, width, step=16)
        def _(j):
            zrows[i, pl.ds(j, 16)] = zv

    @pl.loop(0, ROWS_PER_SUB // ZR)
    def _(j):
        pltpu.sync_copy(zrows, acc_sh.at[pl.ds(sub * ROWS_PER_SUB + j * ZR, ZR)])


def _deg_body(src_hbm, dst_hbm, degp_hbm, ones_v, src_v, dst_v, eff_v, zrows,
              acc_sh, sem):
    c = lax.axis_index("c")
    s = lax.axis_index("s")
    base = (c * NS + s) * E_PER_W

    lanes = lax.iota(jnp.int32, 16)
    one_row = jnp.where(lanes == 0, 1.0, 0.0).astype(jnp.float32)

    @pl.loop(0, DEG_K)
    def _(i):
        ones_v[i, :] = one_row

    _zero_shared(zrows, acc_sh, 16, s)
    plsc.subcore_barrier()

    @pl.loop(0, E_PER_W, step=DEG_K)
    def _(off):
        pltpu.sync_copy(src_hbm.at[pl.ds(base + off, DEG_K)], src_v)
        pltpu.sync_copy(dst_hbm.at[pl.ds(base + off, DEG_K)], dst_v)

        @pl.loop(0, DEG_K, step=16)
        def _(i):
            sv = src_v[pl.ds(i, 16)]
            dv = dst_v[pl.ds(i, 16)]
            eff_v[pl.ds(i, 16)] = jnp.where(sv == dv, N_NODES, dv)

        pltpu.sync_copy(ones_v, acc_sh.at[eff_v], add=True)

    plsc.subcore_barrier()
    pltpu.sync_copy(acc_sh.at[pl.ds(s * ROWS_PER_SUB, ROWS_PER_SUB)],
                    degp_hbm.at[c].at[pl.ds(s * ROWS_PER_SUB, ROWS_PER_SUB)])


@jax.jit
def _deg_pass(src, dst):
    f = pl.kernel(
        _deg_body,
        out_type=jax.ShapeDtypeStruct((NC, N_PAD, 16), jnp.float32),
        mesh=_mesh,
        scratch_types=[
            pltpu.VMEM((DEG_K, 16), jnp.float32),
            pltpu.VMEM((DEG_K,), jnp.int32),
            pltpu.VMEM((DEG_K,), jnp.int32),
            pltpu.VMEM((DEG_K,), jnp.int32),
            pltpu.VMEM((ZR, 16), jnp.float32),
            pltpu.VMEM_SHARED((N_PAD, 16), jnp.float32),
            pltpu.SemaphoreType.DMA,
        ],
    )
    return f(src, dst)


def _msg_body(g_hbm, src_hbm, dst_hbm, accp_hbm, src_v, dst_v, eff_v, rows_v,
              zrows, acc_sh, sem):
    c = lax.axis_index("c")
    s = lax.axis_index("s")
    base = (c * NS + s) * E_PER_W

    _zero_shared(zrows, acc_sh, D, s)
    plsc.subcore_barrier()

    @pl.loop(0, E_PER_W, step=MSG_K)
    def _(off):
        pltpu.sync_copy(src_hbm.at[pl.ds(base + off, MSG_K)], src_v)
        pltpu.sync_copy(dst_hbm.at[pl.ds(base + off, MSG_K)], dst_v)

        @pl.loop(0, MSG_K, step=16)
        def _(i):
            sv = src_v[pl.ds(i, 16)]
            dv = dst_v[pl.ds(i, 16)]
            eff_v[pl.ds(i, 16)] = jnp.where(sv == dv, N_NODES, dv)

        pltpu.async_copy(g_hbm.at[src_v], rows_v, sem).wait()
        pltpu.sync_copy(rows_v, acc_sh.at[eff_v], add=True)

    plsc.subcore_barrier()
    pltpu.sync_copy(acc_sh.at[pl.ds(s * ROWS_PER_SUB, ROWS_PER_SUB)],
                    accp_hbm.at[c].at[pl.ds(s * ROWS_PER_SUB, ROWS_PER_SUB)])


@jax.jit
def _msg_pass(g, src, dst):
    f = pl.kernel(
        _msg_body,
        out_type=jax.ShapeDtypeStruct((NC, N_PAD, D), jnp.float32),
        mesh=_mesh,
        scratch_types=[
            pltpu.VMEM((MSG_K,), jnp.int32),
            pltpu.VMEM((MSG_K,), jnp.int32),
            pltpu.VMEM((MSG_K,), jnp.int32),
            pltpu.VMEM((MSG_K, D), jnp.float32),
            pltpu.VMEM((ZR, D), jnp.float32),
            pltpu.VMEM_SHARED((N_PAD, D), jnp.float32),
            pltpu.SemaphoreType.DMA,
        ],
    )
    return f(g, src, dst)


def _dis_from_degp(degp_ref):
    deg = 1.0 + degp_ref[0, :, 0:1] + degp_ref[1, :, 0:1]
    return lax.rsqrt(deg)


def _stage_a_body(x_ref, w1_ref, degp_ref, g1_ref):
    h = jnp.dot(x_ref[...], w1_ref[...], preferred_element_type=jnp.float32)
    g1_ref[...] = h * _dis_from_degp(degp_ref)


def _stage_b_body(accp_ref, g1_ref, degp_ref, w2_ref, b1_ref, g2_ref):
    dis = _dis_from_degp(degp_ref)
    a = accp_ref[0] + accp_ref[1]
    out1 = jnp.maximum((a + g1_ref[...]) * dis + b1_ref[...], 0.0)
    h2 = jnp.dot(out1, w2_ref[...], preferred_element_type=jnp.float32)
    g2_ref[...] = h2 * dis


def _stage_c_body(accp_ref, g2_ref, degp_ref, b2_ref, out_ref):
    dis = _dis_from_degp(degp_ref)
    a = accp_ref[0] + accp_ref[1]
    out_ref[...] = (a + g2_ref[...]) * dis + b2_ref[...]


_TC_R = 1000

_g_spec = pl.BlockSpec((_TC_R, D), lambda i: (i, 0))
_degp_spec = pl.BlockSpec((NC, _TC_R, 16), lambda i: (0, i, 0))
_accp_spec = pl.BlockSpec((NC, _TC_R, D), lambda i: (0, i, 0))
_w_spec = pl.BlockSpec((D, D), lambda i: (0, 0))
_b_spec = pl.BlockSpec((1, D), lambda i: (0, 0))
_grid = (N_NODES // _TC_R,)
_g_shape = jax.ShapeDtypeStruct((N_NODES, D), jnp.float32)


@jax.jit
def _stage_a(x, w1, degp):
    return pl.pallas_call(
        _stage_a_body,
        grid=_grid,
        in_specs=[_g_spec, _w_spec, _degp_spec],
        out_specs=_g_spec,
        out_shape=_g_shape,
    )(x, w1, degp)


@jax.jit
def _stage_b(accp, g1, degp, w2, b1):
    return pl.pallas_call(
        _stage_b_body,
        grid=_grid,
        in_specs=[_accp_spec, _g_spec, _degp_spec, _w_spec, _b_spec],
        out_specs=_g_spec,
        out_shape=_g_shape,
    )(accp, g1, degp, w2, b1)


@jax.jit
def _stage_c(accp, g2, degp, b2):
    return pl.pallas_call(
        _stage_c_body,
        grid=_grid,
        in_specs=[_accp_spec, _g_spec, _degp_spec, _b_spec],
        out_specs=_g_spec,
        out_shape=_g_shape,
    )(accp, g2, degp, b2)


def kernel(x, edge_index, W1, b1, W2, b2):
    src = edge_index[0].astype(jnp.int32)
    dst = edge_index[1].astype(jnp.int32)
    degp = _deg_pass(src, dst)
    g1 = _stage_a(x, W1, degp)
    acc1 = _msg_pass(g1, src, dst)
    g2 = _stage_b(acc1, g1, degp, W2, b1.reshape(1, D))
    acc2 = _msg_pass(g2, src, dst)
    return _stage_c(acc2, g2, degp, b2.reshape(1, D))


# speedup vs baseline: 20.3856x; 20.3856x over previous
import jax
import jax.numpy as jnp
from jax import lax
from jax.experimental import pallas as pl
from jax.experimental.pallas import tpu as pltpu
from jax.experimental.pallas import tpu_sc as plsc

N_NODES = 10000
E_EDGES = 320000
D = 128
NC, NS = 2, 16
NW = NC * NS
E_PER_W = E_EDGES // NW
N_PAD = 10240
ROWS_PER_SUB = N_PAD // NS
DEG_K = 2000
MSG_K = 200
ZR = 16

_mesh = plsc.VectorSubcoreMesh(core_axis_name="c", subcore_axis_name="s")
_sc_params = pltpu.CompilerParams(use_tc_tiling_on_sc=False)


def _zero_shared(zrows, acc_sh, width, sub):
    zv = jnp.zeros((16,), jnp.float32)

    @pl.loop(0, ZR)
    def _(i):
        @pl.loop(0, width, step=16)
        def _(j):
            zrows[i, pl.ds(j, 16)] = zv

    @pl.loop(0, ROWS_PER_SUB // ZR)
    def _(j):
        pltpu.sync_copy(zrows, acc_sh.at[pl.ds(sub * ROWS_PER_SUB + j * ZR, ZR)])


def _deg_body(src_hbm, dst_hbm, degp_hbm, ones_v, src_v, dst_v, eff_v, zrows,
              acc_sh, sem):
    c = lax.axis_index("c")
    s = lax.axis_index("s")
    base = (c * NS + s) * E_PER_W

    lanes = lax.iota(jnp.int32, 16)
    one_row = jnp.where(lanes == 0, 1.0, 0.0).astype(jnp.float32)

    @pl.loop(0, DEG_K)
    def _(i):
        ones_v[i, :] = one_row

    _zero_shared(zrows, acc_sh, 16, s)
    plsc.subcore_barrier()

    @pl.loop(0, E_PER_W, step=DEG_K)
    def _(off):
        pltpu.sync_copy(src_hbm.at[pl.ds(base + off, DEG_K)], src_v)
        pltpu.sync_copy(dst_hbm.at[pl.ds(base + off, DEG_K)], dst_v)

        @pl.loop(0, DEG_K, step=16)
        def _(i):
            sv = src_v[pl.ds(i, 16)]
            dv = dst_v[pl.ds(i, 16)]
            eff_v[pl.ds(i, 16)] = jnp.where(sv == dv, N_NODES, dv)

        pltpu.sync_copy(ones_v, acc_sh.at[eff_v], add=True)

    plsc.subcore_barrier()
    pltpu.sync_copy(acc_sh.at[pl.ds(s * ROWS_PER_SUB, ROWS_PER_SUB)],
                    degp_hbm.at[c].at[pl.ds(s * ROWS_PER_SUB, ROWS_PER_SUB)])


@jax.jit
def _deg_pass(src, dst):
    f = pl.kernel(
        _deg_body,
        out_type=jax.ShapeDtypeStruct((NC, N_PAD, 16), jnp.float32),
        mesh=_mesh,
        compiler_params=_sc_params,
        scratch_types=[
            pltpu.VMEM((DEG_K, 16), jnp.float32),
            pltpu.VMEM((DEG_K,), jnp.int32),
            pltpu.VMEM((DEG_K,), jnp.int32),
            pltpu.VMEM((DEG_K,), jnp.int32),
            pltpu.VMEM((ZR, 16), jnp.float32),
            pltpu.VMEM_SHARED((N_PAD, 16), jnp.float32),
            pltpu.SemaphoreType.DMA,
        ],
    )
    return f(src, dst)


def _msg_body(g_hbm, src_hbm, dst_hbm, accp_hbm, src_v, dst_v, eff_v, zrows,
              rows_v, acc_sh, sem):
    c = lax.axis_index("c")
    s = lax.axis_index("s")
    base = (c * NS + s) * E_PER_W

    _zero_shared(zrows, acc_sh, D, s)
    plsc.subcore_barrier()

    @pl.loop(0, E_PER_W, step=MSG_K)
    def _(off):
        pltpu.sync_copy(src_hbm.at[pl.ds(base + off, MSG_K)], src_v)
        pltpu.sync_copy(dst_hbm.at[pl.ds(base + off, MSG_K)], dst_v)

        @pl.loop(0, MSG_K, step=16)
        def _(i):
            sv = src_v[pl.ds(i, 16)]
            dv = dst_v[pl.ds(i, 16)]
            eff_v[pl.ds(i, 16)] = jnp.where(sv == dv, N_NODES, dv)

        pltpu.async_copy(g_hbm.at[src_v], rows_v, sem).wait()
        pltpu.sync_copy(rows_v, acc_sh.at[eff_v], add=True)

    plsc.subcore_barrier()
    pltpu.sync_copy(acc_sh.at[pl.ds(s * ROWS_PER_SUB, ROWS_PER_SUB)],
                    accp_hbm.at[c].at[pl.ds(s * ROWS_PER_SUB, ROWS_PER_SUB)])


@jax.jit
def _msg_pass(g, src, dst):
    f = pl.kernel(
        _msg_body,
        out_type=jax.ShapeDtypeStruct((NC, N_PAD, D), jnp.float32),
        mesh=_mesh,
        compiler_params=_sc_params,
        scratch_types=[
            pltpu.VMEM((MSG_K,), jnp.int32),
            pltpu.VMEM((MSG_K,), jnp.int32),
            pltpu.VMEM((MSG_K,), jnp.int32),
            pltpu.VMEM((ZR, D), jnp.float32),
            pltpu.VMEM((MSG_K, D), jnp.float32),
            pltpu.VMEM_SHARED((N_PAD, D), jnp.float32),
            pltpu.SemaphoreType.DMA,
        ],
    )
    return f(g, src, dst)


def _dis_from_degp(degp_ref):
    deg = 1.0 + degp_ref[0, :, 0:1] + degp_ref[1, :, 0:1]
    return lax.rsqrt(deg)


def _stage_a_body(x_ref, w1_ref, degp_ref, g1_ref):
    h = jnp.dot(x_ref[...], w1_ref[...], preferred_element_type=jnp.float32)
    g1_ref[...] = h * _dis_from_degp(degp_ref)


def _stage_b_body(accp_ref, g1_ref, degp_ref, w2_ref, b1_ref, g2_ref):
    dis = _dis_from_degp(degp_ref)
    a = accp_ref[0] + accp_ref[1]
    out1 = jnp.maximum((a + g1_ref[...]) * dis + b1_ref[...], 0.0)
    h2 = jnp.dot(out1, w2_ref[...], preferred_element_type=jnp.float32)
    g2_ref[...] = h2 * dis


def _stage_c_body(accp_ref, g2_ref, degp_ref, b2_ref, out_ref):
    dis = _dis_from_degp(degp_ref)
    a = accp_ref[0] + accp_ref[1]
    out_ref[...] = (a + g2_ref[...]) * dis + b2_ref[...]


_TC_R = 1000

_g_spec = pl.BlockSpec((_TC_R, D), lambda i: (i, 0))
_degp_spec = pl.BlockSpec((NC, _TC_R, 16), lambda i: (0, i, 0))
_accp_spec = pl.BlockSpec((NC, _TC_R, D), lambda i: (0, i, 0))
_w_spec = pl.BlockSpec((D, D), lambda i: (0, 0))
_b_spec = pl.BlockSpec((1, D), lambda i: (0, 0))
_grid = (N_NODES // _TC_R,)
_g_shape = jax.ShapeDtypeStruct((N_NODES, D), jnp.float32)


@jax.jit
def _stage_a(x, w1, degp):
    return pl.pallas_call(
        _stage_a_body,
        grid=_grid,
        in_specs=[_g_spec, _w_spec, _degp_spec],
        out_specs=_g_spec,
        out_shape=_g_shape,
    )(x, w1, degp)


@jax.jit
def _stage_b(accp, g1, degp, w2, b1):
    return pl.pallas_call(
        _stage_b_body,
        grid=_grid,
        in_specs=[_accp_spec, _g_spec, _degp_spec, _w_spec, _b_spec],
        out_specs=_g_spec,
        out_shape=_g_shape,
    )(accp, g1, degp, w2, b1)


@jax.jit
def _stage_c(accp, g2, degp, b2):
    return pl.pallas_call(
        _stage_c_body,
        grid=_grid,
        in_specs=[_accp_spec, _g_spec, _degp_spec, _b_spec],
        out_specs=_g_spec,
        out_shape=_g_shape,
    )(accp, g2, degp, b2)


def kernel(x, edge_index, W1, b1, W2, b2):
    src = edge_index[0].astype(jnp.int32)
    dst = edge_index[1].astype(jnp.int32)
    degp = _deg_pass(src, dst)
    g1 = _stage_a(x, W1, degp)
    acc1 = _msg_pass(g1, src, dst)
    g2 = _stage_b(acc1, g1, degp, W2, b1.reshape(1, D))
    acc2 = _msg_pass(g2, src, dst)
    return _stage_c(acc2, g2, degp, b2.reshape(1, D))


# speedup vs baseline: 21.7655x; 1.0677x over previous
import jax
import jax.numpy as jnp
from jax import lax
from jax.experimental import pallas as pl
from jax.experimental.pallas import tpu as pltpu
from jax.experimental.pallas import tpu_sc as plsc

N_NODES = 10000
E_EDGES = 320000
D = 128
NC, NS = 2, 16
NW = NC * NS
E_PER_W = E_EDGES // NW
N_PAD = 10240
ROWS_PER_SUB = N_PAD // NS
DEG_K = 2000
MSG_K = 80
ZR = 16

_mesh = plsc.VectorSubcoreMesh(core_axis_name="c", subcore_axis_name="s")
_sc_params = pltpu.CompilerParams(use_tc_tiling_on_sc=False)


def _zero_shared(zrows, acc_sh, width, sub):
    zv = jnp.zeros((16,), jnp.float32)

    @pl.loop(0, ZR)
    def _(i):
        @pl.loop(0, width, step=16)
        def _(j):
            zrows[i, pl.ds(j, 16)] = zv

    @pl.loop(0, ROWS_PER_SUB // ZR)
    def _(j):
        pltpu.sync_copy(zrows, acc_sh.at[pl.ds(sub * ROWS_PER_SUB + j * ZR, ZR)])


def _deg_body(src_hbm, dst_hbm, degp_hbm, ones_v, src_v, dst_v, eff_v, zrows,
              acc_sh, sem):
    c = lax.axis_index("c")
    s = lax.axis_index("s")
    base = (c * NS + s) * E_PER_W

    lanes = lax.iota(jnp.int32, 16)
    one_row = jnp.where(lanes == 0, 1.0, 0.0).astype(jnp.float32)

    @pl.loop(0, DEG_K)
    def _(i):
        ones_v[i, :] = one_row

    _zero_shared(zrows, acc_sh, 16, s)
    plsc.subcore_barrier()

    @pl.loop(0, E_PER_W, step=DEG_K)
    def _(off):
        pltpu.sync_copy(src_hbm.at[pl.ds(base + off, DEG_K)], src_v)
        pltpu.sync_copy(dst_hbm.at[pl.ds(base + off, DEG_K)], dst_v)

        @pl.loop(0, DEG_K, step=16)
        def _(i):
            sv = src_v[pl.ds(i, 16)]
            dv = dst_v[pl.ds(i, 16)]
            eff_v[pl.ds(i, 16)] = jnp.where(sv == dv, N_NODES, dv)

        pltpu.sync_copy(ones_v, acc_sh.at[eff_v], add=True)

    plsc.subcore_barrier()
    pltpu.sync_copy(acc_sh.at[pl.ds(s * ROWS_PER_SUB, ROWS_PER_SUB)],
                    degp_hbm.at[c].at[pl.ds(s * ROWS_PER_SUB, ROWS_PER_SUB)])


@jax.jit
def _deg_pass(src, dst):
    f = pl.kernel(
        _deg_body,
        out_type=jax.ShapeDtypeStruct((NC, N_PAD, 16), jnp.float32),
        mesh=_mesh,
        compiler_params=_sc_params,
        scratch_types=[
            pltpu.VMEM((DEG_K, 16), jnp.float32),
            pltpu.VMEM((DEG_K,), jnp.int32),
            pltpu.VMEM((DEG_K,), jnp.int32),
            pltpu.VMEM((DEG_K,), jnp.int32),
            pltpu.VMEM((ZR, 16), jnp.float32),
            pltpu.VMEM_SHARED((N_PAD, 16), jnp.float32),
            pltpu.SemaphoreType.DMA,
        ],
    )
    return f(src, dst)


N_CHUNK = E_PER_W // MSG_K


def _msg_body(g_hbm, src_hbm, dst_hbm, accp_hbm,
              src_a, dst_a, eff_a, rows_a, src_b, dst_b, eff_b, rows_b,
              zrows, acc_sh, gsem_a, gsem_b, ssem_a, ssem_b):
    c = lax.axis_index("c")
    s = lax.axis_index("s")
    base = (c * NS + s) * E_PER_W

    _zero_shared(zrows, acc_sh, D, s)
    plsc.subcore_barrier()

    bufs = ((src_a, dst_a, eff_a, rows_a, gsem_a, ssem_a),
            (src_b, dst_b, eff_b, rows_b, gsem_b, ssem_b))

    def issue(j, b):
        src_v, dst_v, eff_v, rows_v, gsem, _ = bufs[b]
        off = base + j * MSG_K
        pltpu.sync_copy(src_hbm.at[pl.ds(off, MSG_K)], src_v)
        pltpu.sync_copy(dst_hbm.at[pl.ds(off, MSG_K)], dst_v)

        @pl.loop(0, MSG_K, step=16)
        def _(i):
            sv = src_v[pl.ds(i, 16)]
            dv = dst_v[pl.ds(i, 16)]
            eff_v[pl.ds(i, 16)] = jnp.where(sv == dv, N_NODES, dv)

        pltpu.async_copy(g_hbm.at[src_v], rows_v, gsem)

    def proc(b):
        src_v, _, eff_v, rows_v, gsem, ssem = bufs[b]
        pltpu.make_async_copy(g_hbm.at[src_v], rows_v, gsem).wait()
        pltpu.async_copy(rows_v, acc_sh.at[eff_v], ssem, add=True)

    def drain(b):
        _, _, eff_v, rows_v, _, ssem = bufs[b]
        pltpu.make_async_copy(rows_v, acc_sh.at[eff_v], ssem).wait()

    issue(0, 0)

    @pl.loop(0, N_CHUNK - 1, step=2)
    def _(j):
        @pl.when(j > 0)
        def _():
            drain(1)

        issue(j + 1, 1)
        proc(0)

        @pl.when(j + 2 < N_CHUNK)
        def _():
            drain(0)
            issue(j + 2, 0)

        proc(1)

    proc(0)
    drain(0)
    drain(1)

    plsc.subcore_barrier()
    pltpu.sync_copy(acc_sh.at[pl.ds(s * ROWS_PER_SUB, ROWS_PER_SUB)],
                    accp_hbm.at[c].at[pl.ds(s * ROWS_PER_SUB, ROWS_PER_SUB)])


@jax.jit
def _msg_pass(g, src, dst):
    idx_bufs = [pltpu.VMEM((MSG_K,), jnp.int32)] * 3
    f = pl.kernel(
        _msg_body,
        out_type=jax.ShapeDtypeStruct((NC, N_PAD, D), jnp.float32),
        mesh=_mesh,
        compiler_params=_sc_params,
        scratch_types=[
            *idx_bufs,
            pltpu.VMEM((MSG_K, D), jnp.float32),
            *idx_bufs,
            pltpu.VMEM((MSG_K, D), jnp.float32),
            pltpu.VMEM((ZR, D), jnp.float32),
            pltpu.VMEM_SHARED((N_PAD, D), jnp.float32),
            pltpu.SemaphoreType.DMA,
            pltpu.SemaphoreType.DMA,
            pltpu.SemaphoreType.DMA,
            pltpu.SemaphoreType.DMA,
        ],
    )
    return f(g, src, dst)


def _dis_from_degp(degp_ref):
    deg = 1.0 + degp_ref[0, :, 0:1] + degp_ref[1, :, 0:1]
    return lax.rsqrt(deg)


def _stage_a_body(x_ref, w1_ref, degp_ref, g1_ref):
    h = jnp.dot(x_ref[...], w1_ref[...], preferred_element_type=jnp.float32)
    g1_ref[...] = h * _dis_from_degp(degp_ref)


def _stage_b_body(accp_ref, g1_ref, degp_ref, w2_ref, b1_ref, g2_ref):
    dis = _dis_from_degp(degp_ref)
    a = accp_ref[0] + accp_ref[1]
    out1 = jnp.maximum((a + g1_ref[...]) * dis + b1_ref[...], 0.0)
    h2 = jnp.dot(out1, w2_ref[...], preferred_element_type=jnp.float32)
    g2_ref[...] = h2 * dis


def _stage_c_body(accp_ref, g2_ref, degp_ref, b2_ref, out_ref):
    dis = _dis_from_degp(degp_ref)
    a = accp_ref[0] + accp_ref[1]
    out_ref[...] = (a + g2_ref[...]) * dis + b2_ref[...]


_TC_R = 1000

_g_spec = pl.BlockSpec((_TC_R, D), lambda i: (i, 0))
_degp_spec = pl.BlockSpec((NC, _TC_R, 16), lambda i: (0, i, 0))
_accp_spec = pl.BlockSpec((NC, _TC_R, D), lambda i: (0, i, 0))
_w_spec = pl.BlockSpec((D, D), lambda i: (0, 0))
_b_spec = pl.BlockSpec((1, D), lambda i: (0, 0))
_grid = (N_NODES // _TC_R,)
_g_shape = jax.ShapeDtypeStruct((N_NODES, D), jnp.float32)


@jax.jit
def _stage_a(x, w1, degp):
    return pl.pallas_call(
        _stage_a_body,
        grid=_grid,
        in_specs=[_g_spec, _w_spec, _degp_spec],
        out_specs=_g_spec,
        out_shape=_g_shape,
    )(x, w1, degp)


@jax.jit
def _stage_b(accp, g1, degp, w2, b1):
    return pl.pallas_call(
        _stage_b_body,
        grid=_grid,
        in_specs=[_accp_spec, _g_spec, _degp_spec, _w_spec, _b_spec],
        out_specs=_g_spec,
        out_shape=_g_shape,
    )(accp, g1, degp, w2, b1)


@jax.jit
def _stage_c(accp, g2, degp, b2):
    return pl.pallas_call(
        _stage_c_body,
        grid=_grid,
        in_specs=[_accp_spec, _g_spec, _degp_spec, _b_spec],
        out_specs=_g_spec,
        out_shape=_g_shape,
    )(accp, g2, degp, b2)


def kernel(x, edge_index, W1, b1, W2, b2):
    src = edge_index[0].astype(jnp.int32)
    dst = edge_index[1].astype(jnp.int32)
    degp = _deg_pass(src, dst)
    g1 = _stage_a(x, W1, degp)
    acc1 = _msg_pass(g1, src, dst)
    g2 = _stage_b(acc1, g1, degp, W2, b1.reshape(1, D))
    acc2 = _msg_pass(g2, src, dst)
    return _stage_c(acc2, g2, degp, b2.reshape(1, D))


# speedup vs baseline: 29.9137x; 1.3744x over previous
import jax
import jax.numpy as jnp
from jax import lax
from jax.experimental import pallas as pl
from jax.experimental.pallas import tpu as pltpu
from jax.experimental.pallas import tpu_sc as plsc

N_NODES = 10000
E_EDGES = 320000
D = 128
NC, NS = 2, 16
NW = NC * NS
E_PER_W = E_EDGES // NW
N_PAD = 10240
ROWS_PER_SUB = N_PAD // NS
DEG_K = 2000
MSG_K = 80
ZR = 16

_mesh = plsc.VectorSubcoreMesh(core_axis_name="c", subcore_axis_name="s")
_sc_params = pltpu.CompilerParams(use_tc_tiling_on_sc=False)


def _zero_shared(zrows, acc_sh, width, sub):
    zv = jnp.zeros((16,), jnp.float32)

    @pl.loop(0, ZR)
    def _(i):
        @pl.loop(0, width, step=16)
        def _(j):
            zrows[i, pl.ds(j, 16)] = zv

    @pl.loop(0, ROWS_PER_SUB // ZR)
    def _(j):
        pltpu.sync_copy(zrows, acc_sh.at[pl.ds(sub * ROWS_PER_SUB + j * ZR, ZR)])


def _deg_body(src_hbm, dst_hbm, degp_hbm, ones_v, src_v, dst_v, eff_v, zrows,
              acc_sh, sem):
    c = lax.axis_index("c")
    s = lax.axis_index("s")
    base = (c * NS + s) * E_PER_W

    lanes = lax.iota(jnp.int32, 16)
    one_row = jnp.where(lanes == 0, 1.0, 0.0).astype(jnp.float32)

    @pl.loop(0, DEG_K)
    def _(i):
        ones_v[i, :] = one_row

    _zero_shared(zrows, acc_sh, 16, s)
    plsc.subcore_barrier()

    @pl.loop(0, E_PER_W, step=DEG_K)
    def _(off):
        pltpu.sync_copy(src_hbm.at[pl.ds(base + off, DEG_K)], src_v)
        pltpu.sync_copy(dst_hbm.at[pl.ds(base + off, DEG_K)], dst_v)

        @pl.loop(0, DEG_K, step=16)
        def _(i):
            sv = src_v[pl.ds(i, 16)]
            dv = dst_v[pl.ds(i, 16)]
            eff_v[pl.ds(i, 16)] = jnp.where(sv == dv, N_NODES, dv)

        pltpu.sync_copy(ones_v, acc_sh.at[eff_v], add=True)

    plsc.subcore_barrier()
    pltpu.sync_copy(acc_sh.at[pl.ds(s * ROWS_PER_SUB, ROWS_PER_SUB)],
                    degp_hbm.at[c].at[pl.ds(s * ROWS_PER_SUB, ROWS_PER_SUB)])


@jax.jit
def _deg_pass(src, dst):
    f = pl.kernel(
        _deg_body,
        out_type=jax.ShapeDtypeStruct((NC, N_PAD, 16), jnp.float32),
        mesh=_mesh,
        compiler_params=_sc_params,
        scratch_types=[
            pltpu.VMEM((DEG_K, 16), jnp.float32),
            pltpu.VMEM((DEG_K,), jnp.int32),
            pltpu.VMEM((DEG_K,), jnp.int32),
            pltpu.VMEM((DEG_K,), jnp.int32),
            pltpu.VMEM((ZR, 16), jnp.float32),
            pltpu.VMEM_SHARED((N_PAD, 16), jnp.float32),
            pltpu.SemaphoreType.DMA,
        ],
    )
    return f(src, dst)


N_CHUNK = E_PER_W // MSG_K


def _msg_body(g_hbm, src2_hbm, dst2_hbm, accp_hbm,
              srcb, effb, rows_a, rows_b, zrows, acc_sh,
              gsem_a, gsem_b, ssem_a, ssem_b):
    c = lax.axis_index("c")
    s = lax.axis_index("s")
    row0 = (c * NS + s) * N_CHUNK

    pltpu.sync_copy(src2_hbm.at[pl.ds(row0, N_CHUNK)], srcb)
    pltpu.sync_copy(dst2_hbm.at[pl.ds(row0, N_CHUNK)], effb)

    _zero_shared(zrows, acc_sh, D, s)

    @pl.loop(0, N_CHUNK)
    def _(j):
        @pl.loop(0, MSG_K, step=16)
        def _(i):
            sv = srcb[j, pl.ds(i, 16)]
            dv = effb[j, pl.ds(i, 16)]
            effb[j, pl.ds(i, 16)] = jnp.where(sv == dv, N_NODES, dv)

    plsc.subcore_barrier()

    bufs = ((rows_a, gsem_a, ssem_a), (rows_b, gsem_b, ssem_b))

    def issue(j, b):
        rows_v, gsem, _ = bufs[b]
        pltpu.async_copy(g_hbm.at[srcb.at[j]], rows_v, gsem)

    def proc(j, b):
        rows_v, gsem, ssem = bufs[b]
        pltpu.make_async_copy(g_hbm.at[srcb.at[j]], rows_v, gsem).wait()
        pltpu.async_copy(rows_v, acc_sh.at[effb.at[j]], ssem, add=True)

    def drain(j, b):
        rows_v, _, ssem = bufs[b]
        pltpu.make_async_copy(rows_v, acc_sh.at[effb.at[j]], ssem).wait()

    issue(0, 0)

    @pl.loop(0, N_CHUNK - 1, step=2)
    def _(j):
        @pl.when(j > 0)
        def _():
            drain(j - 1, 1)

        issue(j + 1, 1)
        proc(j, 0)

        @pl.when(j + 2 < N_CHUNK)
        def _():
            drain(j, 0)
            issue(j + 2, 0)

        proc(j + 1, 1)

    proc(N_CHUNK - 1, 0)
    drain(N_CHUNK - 1, 0)
    drain(N_CHUNK - 2, 1)

    plsc.subcore_barrier()
    pltpu.sync_copy(acc_sh.at[pl.ds(s * ROWS_PER_SUB, ROWS_PER_SUB)],
                    accp_hbm.at[c].at[pl.ds(s * ROWS_PER_SUB, ROWS_PER_SUB)])


@jax.jit
def _msg_pass(g, src2, dst2):
    f = pl.kernel(
        _msg_body,
        out_type=jax.ShapeDtypeStruct((NC, N_PAD, D), jnp.float32),
        mesh=_mesh,
        compiler_params=_sc_params,
        scratch_types=[
            pltpu.VMEM((N_CHUNK, MSG_K), jnp.int32),
            pltpu.VMEM((N_CHUNK, MSG_K), jnp.int32),
            pltpu.VMEM((MSG_K, D), jnp.float32),
            pltpu.VMEM((MSG_K, D), jnp.float32),
            pltpu.VMEM((ZR, D), jnp.float32),
            pltpu.VMEM_SHARED((N_PAD, D), jnp.float32),
            pltpu.SemaphoreType.DMA,
            pltpu.SemaphoreType.DMA,
            pltpu.SemaphoreType.DMA,
            pltpu.SemaphoreType.DMA,
        ],
    )
    return f(g, src2, dst2)


def _dis_from_degp(degp_ref):
    deg = 1.0 + degp_ref[0, :, 0:1] + degp_ref[1, :, 0:1]
    return lax.rsqrt(deg)


def _stage_a_body(x_ref, w1_ref, degp_ref, g1_ref):
    h = jnp.dot(x_ref[...], w1_ref[...], preferred_element_type=jnp.float32)
    g1_ref[...] = h * _dis_from_degp(degp_ref)


def _stage_b_body(accp_ref, g1_ref, degp_ref, w2_ref, b1_ref, g2_ref):
    dis = _dis_from_degp(degp_ref)
    a = accp_ref[0] + accp_ref[1]
    out1 = jnp.maximum((a + g1_ref[...]) * dis + b1_ref[...], 0.0)
    h2 = jnp.dot(out1, w2_ref[...], preferred_element_type=jnp.float32)
    g2_ref[...] = h2 * dis


def _stage_c_body(accp_ref, g2_ref, degp_ref, b2_ref, out_ref):
    dis = _dis_from_degp(degp_ref)
    a = accp_ref[0] + accp_ref[1]
    out_ref[...] = (a + g2_ref[...]) * dis + b2_ref[...]


_TC_R = 1000

_g_spec = pl.BlockSpec((_TC_R, D), lambda i: (i, 0))
_degp_spec = pl.BlockSpec((NC, _TC_R, 16), lambda i: (0, i, 0))
_accp_spec = pl.BlockSpec((NC, _TC_R, D), lambda i: (0, i, 0))
_w_spec = pl.BlockSpec((D, D), lambda i: (0, 0))
_b_spec = pl.BlockSpec((1, D), lambda i: (0, 0))
_grid = (N_NODES // _TC_R,)
_g_shape = jax.ShapeDtypeStruct((N_NODES, D), jnp.float32)


@jax.jit
def _stage_a(x, w1, degp):
    return pl.pallas_call(
        _stage_a_body,
        grid=_grid,
        in_specs=[_g_spec, _w_spec, _degp_spec],
        out_specs=_g_spec,
        out_shape=_g_shape,
    )(x, w1, degp)


@jax.jit
def _stage_b(accp, g1, degp, w2, b1):
    return pl.pallas_call(
        _stage_b_body,
        grid=_grid,
        in_specs=[_accp_spec, _g_spec, _degp_spec, _w_spec, _b_spec],
        out_specs=_g_spec,
        out_shape=_g_shape,
    )(accp, g1, degp, w2, b1)


@jax.jit
def _stage_c(accp, g2, degp, b2):
    return pl.pallas_call(
        _stage_c_body,
        grid=_grid,
        in_specs=[_accp_spec, _g_spec, _degp_spec, _b_spec],
        out_specs=_g_spec,
        out_shape=_g_shape,
    )(accp, g2, degp, b2)


def kernel(x, edge_index, W1, b1, W2, b2):
    src = edge_index[0].astype(jnp.int32)
    dst = edge_index[1].astype(jnp.int32)
    src2 = src.reshape(E_EDGES // MSG_K, MSG_K)
    dst2 = dst.reshape(E_EDGES // MSG_K, MSG_K)
    degp = _deg_pass(src, dst)
    g1 = _stage_a(x, W1, degp)
    acc1 = _msg_pass(g1, src2, dst2)
    g2 = _stage_b(acc1, g1, degp, W2, b1.reshape(1, D))
    acc2 = _msg_pass(g2, src2, dst2)
    return _stage_c(acc2, g2, degp, b2.reshape(1, D))


# speedup vs baseline: 31.2696x; 1.0453x over previous
import jax
import jax.numpy as jnp
from jax import lax
from jax.experimental import pallas as pl
from jax.experimental.pallas import tpu as pltpu
from jax.experimental.pallas import tpu_sc as plsc

N_NODES = 10000
E_EDGES = 320000
D = 128
NC, NS = 2, 16
NW = NC * NS
E_PER_W = E_EDGES // NW
N_PAD = 10240
ROWS_PER_SUB = N_PAD // NS
DEG_K = 2000
MSG_K = 80
ZR = 16

_mesh = plsc.VectorSubcoreMesh(core_axis_name="c", subcore_axis_name="s")
_sc_params = pltpu.CompilerParams(use_tc_tiling_on_sc=False)


def _zero_shared(zrows, acc_sh, width, sub):
    zv = jnp.zeros((16,), jnp.float32)

    @pl.loop(0, ZR)
    def _(i):
        @pl.loop(0, width, step=16)
        def _(j):
            zrows[i, pl.ds(j, 16)] = zv

    @pl.loop(0, ROWS_PER_SUB // ZR)
    def _(j):
        pltpu.sync_copy(zrows, acc_sh.at[pl.ds(sub * ROWS_PER_SUB + j * ZR, ZR)])


N_CHUNK = E_PER_W // MSG_K


def _idx_preload(src2_hbm, dst2_hbm, srcb, effb, row0):
    pltpu.sync_copy(src2_hbm.at[pl.ds(row0, N_CHUNK)], srcb)
    pltpu.sync_copy(dst2_hbm.at[pl.ds(row0, N_CHUNK)], effb)

    @pl.loop(0, N_CHUNK)
    def _(j):
        @pl.loop(0, MSG_K, step=16)
        def _(i):
            sv = srcb[j, pl.ds(i, 16)]
            dv = effb[j, pl.ds(i, 16)]
            effb[j, pl.ds(i, 16)] = jnp.where(sv == dv, N_NODES, dv)


def _deg_body(src2_hbm, dst2_hbm, degp_hbm, srcb, effb, ones_v, zrows,
              acc_sh, sem):
    c = lax.axis_index("c")
    s = lax.axis_index("s")
    row0 = (c * NS + s) * N_CHUNK

    _idx_preload(src2_hbm, dst2_hbm, srcb, effb, row0)

    lanes = lax.iota(jnp.int32, 16)
    one_row = jnp.where(lanes == 0, 1.0, 0.0).astype(jnp.float32)

    @pl.loop(0, MSG_K)
    def _(i):
        ones_v[i, :] = one_row

    _zero_shared(zrows, acc_sh, 16, s)
    plsc.subcore_barrier()

    @pl.loop(0, N_CHUNK)
    def _(j):
        pltpu.async_copy(ones_v, acc_sh.at[effb.at[j]], sem, add=True)

    @pl.loop(0, N_CHUNK)
    def _(j):
        pltpu.make_async_copy(ones_v, acc_sh.at[effb.at[j]], sem).wait()

    plsc.subcore_barrier()
    pltpu.sync_copy(acc_sh.at[pl.ds(s * ROWS_PER_SUB, ROWS_PER_SUB)],
                    degp_hbm.at[c].at[pl.ds(s * ROWS_PER_SUB, ROWS_PER_SUB)])


@jax.jit
def _deg_pass(src2, dst2):
    f = pl.kernel(
        _deg_body,
        out_type=jax.ShapeDtypeStruct((NC, N_PAD, 16), jnp.float32),
        mesh=_mesh,
        compiler_params=_sc_params,
        scratch_types=[
            pltpu.VMEM((N_CHUNK, MSG_K), jnp.int32),
            pltpu.VMEM((N_CHUNK, MSG_K), jnp.int32),
            pltpu.VMEM((MSG_K, 16), jnp.float32),
            pltpu.VMEM((ZR, 16), jnp.float32),
            pltpu.VMEM_SHARED((N_PAD, 16), jnp.float32),
            pltpu.SemaphoreType.DMA,
        ],
    )
    return f(src2, dst2)


def _msg_body(g_hbm, src2_hbm, dst2_hbm, accp_hbm,
              srcb, effb, rows_a, rows_b, zrows, acc_sh,
              gsem_a, gsem_b, ssem_a, ssem_b):
    c = lax.axis_index("c")
    s = lax.axis_index("s")
    row0 = (c * NS + s) * N_CHUNK

    _idx_preload(src2_hbm, dst2_hbm, srcb, effb, row0)
    _zero_shared(zrows, acc_sh, D, s)
    plsc.subcore_barrier()

    bufs = ((rows_a, gsem_a, ssem_a), (rows_b, gsem_b, ssem_b))

    def issue(j, b):
        rows_v, gsem, _ = bufs[b]
        pltpu.async_copy(g_hbm.at[srcb.at[j]], rows_v, gsem)

    def proc(j, b):
        rows_v, gsem, ssem = bufs[b]
        pltpu.make_async_copy(g_hbm.at[srcb.at[j]], rows_v, gsem).wait()
        pltpu.async_copy(rows_v, acc_sh.at[effb.at[j]], ssem, add=True)

    def drain(j, b):
        rows_v, _, ssem = bufs[b]
        pltpu.make_async_copy(rows_v, acc_sh.at[effb.at[j]], ssem).wait()

    issue(0, 0)

    @pl.loop(0, N_CHUNK - 1, step=2)
    def _(j):
        @pl.when(j > 0)
        def _():
            drain(j - 1, 1)

        issue(j + 1, 1)
        proc(j, 0)

        @pl.when(j + 2 < N_CHUNK)
        def _():
            drain(j, 0)
            issue(j + 2, 0)

        proc(j + 1, 1)

    proc(N_CHUNK - 1, 0)
    drain(N_CHUNK - 1, 0)
    drain(N_CHUNK - 2, 1)

    plsc.subcore_barrier()
    pltpu.sync_copy(acc_sh.at[pl.ds(s * ROWS_PER_SUB, ROWS_PER_SUB)],
                    accp_hbm.at[c].at[pl.ds(s * ROWS_PER_SUB, ROWS_PER_SUB)])


@jax.jit
def _msg_pass(g, src2, dst2):
    f = pl.kernel(
        _msg_body,
        out_type=jax.ShapeDtypeStruct((NC, N_PAD, D), jnp.float32),
        mesh=_mesh,
        compiler_params=_sc_params,
        scratch_types=[
            pltpu.VMEM((N_CHUNK, MSG_K), jnp.int32),
            pltpu.VMEM((N_CHUNK, MSG_K), jnp.int32),
            pltpu.VMEM((MSG_K, D), jnp.float32),
            pltpu.VMEM((MSG_K, D), jnp.float32),
            pltpu.VMEM((ZR, D), jnp.float32),
            pltpu.VMEM_SHARED((N_PAD, D), jnp.float32),
            pltpu.SemaphoreType.DMA,
            pltpu.SemaphoreType.DMA,
            pltpu.SemaphoreType.DMA,
            pltpu.SemaphoreType.DMA,
        ],
    )
    return f(g, src2, dst2)


def _dis_from_degp(degp_ref):
    deg = 1.0 + degp_ref[0, :, 0:1] + degp_ref[1, :, 0:1]
    return lax.rsqrt(deg)


def _stage_h_body(x_ref, w1_ref, h1_ref):
    h1_ref[...] = jnp.dot(x_ref[...], w1_ref[...],
                          preferred_element_type=jnp.float32)


def _stage_scale_body(h1_ref, degp_ref, g1_ref):
    g1_ref[...] = h1_ref[...] * _dis_from_degp(degp_ref)


def _stage_b_body(accp_ref, g1_ref, degp_ref, w2_ref, b1_ref, g2_ref):
    dis = _dis_from_degp(degp_ref)
    a = accp_ref[0] + accp_ref[1]
    out1 = jnp.maximum((a + g1_ref[...]) * dis + b1_ref[...], 0.0)
    h2 = jnp.dot(out1, w2_ref[...], preferred_element_type=jnp.float32)
    g2_ref[...] = h2 * dis


def _stage_c_body(accp_ref, g2_ref, degp_ref, b2_ref, out_ref):
    dis = _dis_from_degp(degp_ref)
    a = accp_ref[0] + accp_ref[1]
    out_ref[...] = (a + g2_ref[...]) * dis + b2_ref[...]


_TC_R = 1000

_g_spec = pl.BlockSpec((_TC_R, D), lambda i: (i, 0))
_degp_spec = pl.BlockSpec((NC, _TC_R, 16), lambda i: (0, i, 0))
_accp_spec = pl.BlockSpec((NC, _TC_R, D), lambda i: (0, i, 0))
_w_spec = pl.BlockSpec((D, D), lambda i: (0, 0))
_b_spec = pl.BlockSpec((1, D), lambda i: (0, 0))
_grid = (N_NODES // _TC_R,)
_g_shape = jax.ShapeDtypeStruct((N_NODES, D), jnp.float32)


@jax.jit
def _stage_h(x, w1):
    return pl.pallas_call(
        _stage_h_body,
        grid=_grid,
        in_specs=[_g_spec, _w_spec],
        out_specs=_g_spec,
        out_shape=_g_shape,
    )(x, w1)


@jax.jit
def _stage_scale(h1, degp):
    return pl.pallas_call(
        _stage_scale_body,
        grid=_grid,
        in_specs=[_g_spec, _degp_spec],
        out_specs=_g_spec,
        out_shape=_g_shape,
    )(h1, degp)


@jax.jit
def _stage_b(accp, g1, degp, w2, b1):
    return pl.pallas_call(
        _stage_b_body,
        grid=_grid,
        in_specs=[_accp_spec, _g_spec, _degp_spec, _w_spec, _b_spec],
        out_specs=_g_spec,
        out_shape=_g_shape,
    )(accp, g1, degp, w2, b1)


@jax.jit
def _stage_c(accp, g2, degp, b2):
    return pl.pallas_call(
        _stage_c_body,
        grid=_grid,
        in_specs=[_accp_spec, _g_spec, _degp_spec, _b_spec],
        out_specs=_g_spec,
        out_shape=_g_shape,
    )(accp, g2, degp, b2)


def kernel(x, edge_index, W1, b1, W2, b2):
    src = edge_index[0].astype(jnp.int32)
    dst = edge_index[1].astype(jnp.int32)
    src2 = src.reshape(E_EDGES // MSG_K, MSG_K)
    dst2 = dst.reshape(E_EDGES // MSG_K, MSG_K)
    degp = _deg_pass(src2, dst2)
    h1 = _stage_h(x, W1)
    g1 = _stage_scale(h1, degp)
    acc1 = _msg_pass(g1, src2, dst2)
    g2 = _stage_b(acc1, g1, degp, W2, b1.reshape(1, D))
    acc2 = _msg_pass(g2, src2, dst2)
    return _stage_c(acc2, g2, degp, b2.reshape(1, D))


# speedup vs baseline: 34.5350x; 1.1044x over previous
import jax
import jax.numpy as jnp
from jax import lax
from jax.experimental import pallas as pl
from jax.experimental.pallas import tpu as pltpu
from jax.experimental.pallas import tpu_sc as plsc

N_NODES = 10000
E_EDGES = 320000
D = 128
NC, NS = 2, 16
NW = NC * NS
E_PER_W = E_EDGES // NW
N_PAD = 10016
ROWS_PER_SUB = N_PAD // NS
MSG_K = 80
ZR = 8

_mesh = plsc.VectorSubcoreMesh(core_axis_name="c", subcore_axis_name="s")
_sc_params = pltpu.CompilerParams(use_tc_tiling_on_sc=False)


def _zero_shared(zrows, acc_sh, width, sub):
    zv = jnp.zeros((16,), jnp.float32)

    @pl.loop(0, ZR)
    def _(i):
        @pl.loop(0, width, step=16)
        def _(j):
            zrows[i, pl.ds(j, 16)] = zv

    @pl.loop(0, ROWS_PER_SUB // ZR)
    def _(j):
        pltpu.sync_copy(zrows, acc_sh.at[pl.ds(sub * ROWS_PER_SUB + j * ZR, ZR)])

    rem = ROWS_PER_SUB % ZR
    if rem:
        pltpu.sync_copy(
            zrows.at[pl.ds(0, rem)],
            acc_sh.at[pl.ds(sub * ROWS_PER_SUB + ROWS_PER_SUB - rem, rem)])


N_CHUNK = E_PER_W // MSG_K


def _idx_preload(src2_hbm, dst2_hbm, srcb, effb, row0):
    pltpu.sync_copy(src2_hbm.at[pl.ds(row0, N_CHUNK)], srcb)
    pltpu.sync_copy(dst2_hbm.at[pl.ds(row0, N_CHUNK)], effb)

    @pl.loop(0, N_CHUNK)
    def _(j):
        @pl.loop(0, MSG_K, step=16)
        def _(i):
            sv = srcb[j, pl.ds(i, 16)]
            dv = effb[j, pl.ds(i, 16)]
            effb[j, pl.ds(i, 16)] = jnp.where(sv == dv, N_NODES, dv)


def _deg_body(src2_hbm, dst2_hbm, degp_hbm, srcb, effb, ones_v, zrows,
              acc_sh, sem):
    c = lax.axis_index("c")
    s = lax.axis_index("s")
    row0 = (c * NS + s) * N_CHUNK

    _idx_preload(src2_hbm, dst2_hbm, srcb, effb, row0)

    lanes = lax.iota(jnp.int32, 16)
    one_row = jnp.where(lanes == 0, 1.0, 0.0).astype(jnp.float32)

    @pl.loop(0, MSG_K)
    def _(i):
        ones_v[i, :] = one_row

    _zero_shared(zrows, acc_sh, 16, s)
    plsc.subcore_barrier()

    @pl.loop(0, N_CHUNK)
    def _(j):
        pltpu.async_copy(ones_v, acc_sh.at[effb.at[j]], sem, add=True)

    @pl.loop(0, N_CHUNK)
    def _(j):
        pltpu.make_async_copy(ones_v, acc_sh.at[effb.at[j]], sem).wait()

    plsc.subcore_barrier()
    pltpu.sync_copy(acc_sh.at[pl.ds(s * ROWS_PER_SUB, ROWS_PER_SUB)],
                    degp_hbm.at[c].at[pl.ds(s * ROWS_PER_SUB, ROWS_PER_SUB)])


@jax.jit
def _deg_pass(src2, dst2):
    f = pl.kernel(
        _deg_body,
        out_type=jax.ShapeDtypeStruct((NC, N_PAD, 16), jnp.float32),
        mesh=_mesh,
        compiler_params=_sc_params,
        scratch_types=[
            pltpu.VMEM((N_CHUNK, MSG_K), jnp.int32),
            pltpu.VMEM((N_CHUNK, MSG_K), jnp.int32),
            pltpu.VMEM((MSG_K, 16), jnp.float32),
            pltpu.VMEM((ZR, 16), jnp.float32),
            pltpu.VMEM_SHARED((N_PAD, 16), jnp.float32),
            pltpu.SemaphoreType.DMA,
        ],
    )
    return f(src2, dst2)


def _msg_body(g_hbm, src2_hbm, dst2_hbm, accp_hbm,
              srcb, effb, rows_a, rows_b, rows_c, acc_sh,
              gsem_a, gsem_b, gsem_c, ssem_a, ssem_b, ssem_c):
    c = lax.axis_index("c")
    s = lax.axis_index("s")
    row0 = (c * NS + s) * N_CHUNK

    _idx_preload(src2_hbm, dst2_hbm, srcb, effb, row0)
    _zero_shared(rows_a.at[pl.ds(0, ZR)], acc_sh, D, s)
    plsc.subcore_barrier()

    bufs = ((rows_a, gsem_a, ssem_a),
            (rows_b, gsem_b, ssem_b),
            (rows_c, gsem_c, ssem_c))

    def issue(j, b):
        rows_v, gsem, _ = bufs[b]
        pltpu.async_copy(g_hbm.at[srcb.at[j]], rows_v, gsem)

    def proc(j, b):
        rows_v, gsem, ssem = bufs[b]
        pltpu.make_async_copy(g_hbm.at[srcb.at[j]], rows_v, gsem).wait()
        pltpu.async_copy(rows_v, acc_sh.at[effb.at[j]], ssem, add=True)

    def drain(j, b):
        rows_v, _, ssem = bufs[b]
        pltpu.make_async_copy(rows_v, acc_sh.at[effb.at[j]], ssem).wait()

    def slot(j, drain_behind=True, gather_ahead=True):
        if drain_behind:
            drain(j - 2, (j + 1) % 3)
        if gather_ahead:
            issue(j + 1, (j + 1) % 3)
        proc(j, j % 3)

    issue(0, 0)
    slot(0, drain_behind=False)
    slot(1, drain_behind=False)

    @pl.loop(2, N_CHUNK - 3, step=3)
    def _(j):
        jj = j
        for k in range(3):
            b1 = (2 + k) % 3
            _slot_static(jj + k, b1, bufs, g_hbm, srcb, effb, acc_sh)

    slot(N_CHUNK - 3)
    slot(N_CHUNK - 2)
    slot(N_CHUNK - 1, gather_ahead=False)
    drain(N_CHUNK - 2, (N_CHUNK - 2) % 3)
    drain(N_CHUNK - 1, (N_CHUNK - 1) % 3)

    plsc.subcore_barrier()
    pltpu.sync_copy(acc_sh.at[pl.ds(s * ROWS_PER_SUB, ROWS_PER_SUB)],
                    accp_hbm.at[c].at[pl.ds(s * ROWS_PER_SUB, ROWS_PER_SUB)])


def _slot_static(j, b, bufs, g_hbm, srcb, effb, acc_sh):
    rows_d, _, ssem_d = bufs[(b + 1) % 3]
    pltpu.make_async_copy(rows_d, acc_sh.at[effb.at[j - 2]], ssem_d).wait()
    rows_g, gsem_g, _ = bufs[(b + 1) % 3]
    pltpu.async_copy(g_hbm.at[srcb.at[j + 1]], rows_g, gsem_g)
    rows_v, gsem, ssem = bufs[b]
    pltpu.make_async_copy(g_hbm.at[srcb.at[j]], rows_v, gsem).wait()
    pltpu.async_copy(rows_v, acc_sh.at[effb.at[j]], ssem, add=True)


@jax.jit
def _msg_pass(g, src2, dst2):
    f = pl.kernel(
        _msg_body,
        out_type=jax.ShapeDtypeStruct((NC, N_PAD, D), jnp.float32),
        mesh=_mesh,
        compiler_params=_sc_params,
        scratch_types=[
            pltpu.VMEM((N_CHUNK, MSG_K), jnp.int32),
            pltpu.VMEM((N_CHUNK, MSG_K), jnp.int32),
            pltpu.VMEM((MSG_K, D), jnp.float32),
            pltpu.VMEM((MSG_K, D), jnp.float32),
            pltpu.VMEM((MSG_K, D), jnp.float32),
            pltpu.VMEM_SHARED((N_PAD, D), jnp.float32),
            pltpu.SemaphoreType.DMA,
            pltpu.SemaphoreType.DMA,
            pltpu.SemaphoreType.DMA,
            pltpu.SemaphoreType.DMA,
            pltpu.SemaphoreType.DMA,
            pltpu.SemaphoreType.DMA,
        ],
    )
    return f(g, src2, dst2)


def _dis_from_degp(degp_ref):
    deg = 1.0 + degp_ref[0, :, 0:1] + degp_ref[1, :, 0:1]
    return lax.rsqrt(deg)


def _stage_h_body(x_ref, w1_ref, h1_ref):
    h1_ref[...] = jnp.dot(x_ref[...], w1_ref[...],
                          preferred_element_type=jnp.float32)


def _stage_scale_body(h1_ref, degp_ref, g1_ref):
    g1_ref[...] = h1_ref[...] * _dis_from_degp(degp_ref)


def _stage_b_body(accp_ref, g1_ref, degp_ref, w2_ref, b1_ref, g2_ref):
    dis = _dis_from_degp(degp_ref)
    a = accp_ref[0] + accp_ref[1]
    out1 = jnp.maximum((a + g1_ref[...]) * dis + b1_ref[...], 0.0)
    h2 = jnp.dot(out1, w2_ref[...], preferred_element_type=jnp.float32)
    g2_ref[...] = h2 * dis


def _stage_c_body(accp_ref, g2_ref, degp_ref, b2_ref, out_ref):
    dis = _dis_from_degp(degp_ref)
    a = accp_ref[0] + accp_ref[1]
    out_ref[...] = (a + g2_ref[...]) * dis + b2_ref[...]


_TC_R = 1000

_g_spec = pl.BlockSpec((_TC_R, D), lambda i: (i, 0))
_degp_spec = pl.BlockSpec((NC, _TC_R, 16), lambda i: (0, i, 0))
_accp_spec = pl.BlockSpec((NC, _TC_R, D), lambda i: (0, i, 0))
_w_spec = pl.BlockSpec((D, D), lambda i: (0, 0))
_b_spec = pl.BlockSpec((1, D), lambda i: (0, 0))
_grid = (N_NODES // _TC_R,)
_g_shape = jax.ShapeDtypeStruct((N_NODES, D), jnp.float32)


@jax.jit
def _stage_h(x, w1):
    return pl.pallas_call(
        _stage_h_body,
        grid=_grid,
        in_specs=[_g_spec, _w_spec],
        out_specs=_g_spec,
        out_shape=_g_shape,
    )(x, w1)


@jax.jit
def _stage_scale(h1, degp):
    return pl.pallas_call(
        _stage_scale_body,
        grid=_grid,
        in_specs=[_g_spec, _degp_spec],
        out_specs=_g_spec,
        out_shape=_g_shape,
    )(h1, degp)


@jax.jit
def _stage_b(accp, g1, degp, w2, b1):
    return pl.pallas_call(
        _stage_b_body,
        grid=_grid,
        in_specs=[_accp_spec, _g_spec, _degp_spec, _w_spec, _b_spec],
        out_specs=_g_spec,
        out_shape=_g_shape,
    )(accp, g1, degp, w2, b1)


@jax.jit
def _stage_c(accp, g2, degp, b2):
    return pl.pallas_call(
        _stage_c_body,
        grid=_grid,
        in_specs=[_accp_spec, _g_spec, _degp_spec, _b_spec],
        out_specs=_g_spec,
        out_shape=_g_shape,
    )(accp, g2, degp, b2)


def kernel(x, edge_index, W1, b1, W2, b2):
    src = edge_index[0].astype(jnp.int32)
    dst = edge_index[1].astype(jnp.int32)
    src2 = src.reshape(E_EDGES // MSG_K, MSG_K)
    dst2 = dst.reshape(E_EDGES // MSG_K, MSG_K)
    degp = _deg_pass(src2, dst2)
    h1 = _stage_h(x, W1)
    g1 = _stage_scale(h1, degp)
    acc1 = _msg_pass(g1, src2, dst2)
    g2 = _stage_b(acc1, g1, degp, W2, b1.reshape(1, D))
    acc2 = _msg_pass(g2, src2, dst2)
    return _stage_c(acc2, g2, degp, b2.reshape(1, D))


# speedup vs baseline: 37.4036x; 1.0831x over previous
import jax
import jax.numpy as jnp
from jax import lax
from jax.experimental import pallas as pl
from jax.experimental.pallas import tpu as pltpu
from jax.experimental.pallas import tpu_sc as plsc

N_NODES = 10000
E_EDGES = 320000
D = 128
NC, NS = 2, 16
NW = NC * NS
E_PER_W = E_EDGES // NW
N_PAD = 10016
ROWS_PER_SUB = N_PAD // NS
MSG_K = 80

_mesh = plsc.VectorSubcoreMesh(core_axis_name="c", subcore_axis_name="s")
_sc_params = pltpu.CompilerParams(use_tc_tiling_on_sc=False)


def _zero_vmem(zrows, nrows, width):
    zv = jnp.zeros((16,), jnp.float32)

    @pl.loop(0, nrows)
    def _(i):
        @pl.loop(0, width, step=16)
        def _(j):
            zrows[i, pl.ds(j, 16)] = zv


def _zero_shared(zrows, znrows, acc_sh, sub, zsem):
    base = sub * ROWS_PER_SUB
    nfull = ROWS_PER_SUB // znrows
    rem = ROWS_PER_SUB % znrows

    @pl.loop(0, nfull)
    def _(j):
        pltpu.async_copy(zrows, acc_sh.at[pl.ds(base + j * znrows, znrows)],
                         zsem)

    if rem:
        pltpu.async_copy(zrows.at[pl.ds(0, rem)],
                         acc_sh.at[pl.ds(base + nfull * znrows, rem)], zsem)

    @pl.loop(0, nfull)
    def _(j):
        pltpu.make_async_copy(
            zrows, acc_sh.at[pl.ds(base + j * znrows, znrows)], zsem).wait()

    if rem:
        pltpu.make_async_copy(
            zrows.at[pl.ds(0, rem)],
            acc_sh.at[pl.ds(base + nfull * znrows, rem)], zsem).wait()


N_CHUNK = E_PER_W // MSG_K


def _deg_body(edge2_hbm, degp_hbm, eff2_hbm, srcb, effb, ones_v, zrows,
              acc_sh, sem, zsem, esem):
    c = lax.axis_index("c")
    s = lax.axis_index("s")
    row0 = (c * NS + s) * N_CHUNK

    pltpu.sync_copy(edge2_hbm.at[0].at[pl.ds(row0, N_CHUNK)], srcb)
    pltpu.sync_copy(edge2_hbm.at[1].at[pl.ds(row0, N_CHUNK)], effb)

    @pl.loop(0, N_CHUNK)
    def _(j):
        @pl.loop(0, MSG_K, step=16)
        def _(i):
            sv = srcb[j, pl.ds(i, 16)]
            dv = effb[j, pl.ds(i, 16)]
            effb[j, pl.ds(i, 16)] = jnp.where(sv == dv, N_NODES, dv)

    pltpu.async_copy(effb, eff2_hbm.at[pl.ds(row0, N_CHUNK)], esem)

    lanes = lax.iota(jnp.int32, 16)
    one_row = jnp.where(lanes == 0, 1.0, 0.0).astype(jnp.float32)

    @pl.loop(0, MSG_K)
    def _(i):
        ones_v[i, :] = one_row

    _zero_vmem(zrows, ZDEG, 16)
    _zero_shared(zrows, ZDEG, acc_sh, s, zsem)
    plsc.subcore_barrier()

    @pl.loop(0, N_CHUNK)
    def _(j):
        pltpu.async_copy(ones_v, acc_sh.at[effb.at[j]], sem, add=True)

    @pl.loop(0, N_CHUNK)
    def _(j):
        pltpu.make_async_copy(ones_v, acc_sh.at[effb.at[j]], sem).wait()

    pltpu.make_async_copy(effb, eff2_hbm.at[pl.ds(row0, N_CHUNK)], esem).wait()

    plsc.subcore_barrier()
    pltpu.sync_copy(acc_sh.at[pl.ds(s * ROWS_PER_SUB, ROWS_PER_SUB)],
                    degp_hbm.at[c].at[pl.ds(s * ROWS_PER_SUB, ROWS_PER_SUB)])


ZDEG = 128


@jax.jit
def _deg_pass(edge2):
    f = pl.kernel(
        _deg_body,
        out_type=(jax.ShapeDtypeStruct((NC, N_PAD, 16), jnp.float32),
                  jax.ShapeDtypeStruct((E_EDGES // MSG_K, MSG_K), jnp.int32)),
        mesh=_mesh,
        compiler_params=_sc_params,
        scratch_types=[
            pltpu.VMEM((N_CHUNK, MSG_K), jnp.int32),
            pltpu.VMEM((N_CHUNK, MSG_K), jnp.int32),
            pltpu.VMEM((MSG_K, 16), jnp.float32),
            pltpu.VMEM((ZDEG, 16), jnp.float32),
            pltpu.VMEM_SHARED((N_PAD, 16), jnp.float32),
            pltpu.SemaphoreType.DMA,
            pltpu.SemaphoreType.DMA,
            pltpu.SemaphoreType.DMA,
        ],
    )
    return f(edge2)


def _msg_body(g_hbm, edge2_hbm, eff2_hbm, accp_hbm,
              srcb, effb, rows_a, rows_b, rows_c, acc_sh,
              gsem_a, gsem_b, gsem_c, ssem_a, ssem_b, ssem_c):
    c = lax.axis_index("c")
    s = lax.axis_index("s")
    row0 = (c * NS + s) * N_CHUNK

    pltpu.sync_copy(edge2_hbm.at[0].at[pl.ds(row0, N_CHUNK)], srcb)
    pltpu.sync_copy(eff2_hbm.at[pl.ds(row0, N_CHUNK)], effb)
    _zero_vmem(rows_a, MSG_K, D)
    _zero_shared(rows_a, MSG_K, acc_sh, s, gsem_a)
    plsc.subcore_barrier()

    bufs = ((rows_a, gsem_a, ssem_a),
            (rows_b, gsem_b, ssem_b),
            (rows_c, gsem_c, ssem_c))

    def issue(j, b):
        rows_v, gsem, _ = bufs[b]
        pltpu.async_copy(g_hbm.at[srcb.at[j]], rows_v, gsem)

    def proc(j, b):
        rows_v, gsem, ssem = bufs[b]
        pltpu.make_async_copy(g_hbm.at[srcb.at[j]], rows_v, gsem).wait()
        pltpu.async_copy(rows_v, acc_sh.at[effb.at[j]], ssem, add=True)

    def drain(j, b):
        rows_v, _, ssem = bufs[b]
        pltpu.make_async_copy(rows_v, acc_sh.at[effb.at[j]], ssem).wait()

    def slot(j, drain_behind=True, gather_ahead=True):
        if drain_behind:
            drain(j - 2, (j + 1) % 3)
        if gather_ahead:
            issue(j + 1, (j + 1) % 3)
        proc(j, j % 3)

    issue(0, 0)
    slot(0, drain_behind=False)
    slot(1, drain_behind=False)

    @pl.loop(2, N_CHUNK - 3, step=3)
    def _(j):
        jj = j
        for k in range(3):
            b1 = (2 + k) % 3
            _slot_static(jj + k, b1, bufs, g_hbm, srcb, effb, acc_sh)

    slot(N_CHUNK - 3)
    slot(N_CHUNK - 2)
    slot(N_CHUNK - 1, gather_ahead=False)
    drain(N_CHUNK - 2, (N_CHUNK - 2) % 3)
    drain(N_CHUNK - 1, (N_CHUNK - 1) % 3)

    plsc.subcore_barrier()
    pltpu.sync_copy(acc_sh.at[pl.ds(s * ROWS_PER_SUB, ROWS_PER_SUB)],
                    accp_hbm.at[c].at[pl.ds(s * ROWS_PER_SUB, ROWS_PER_SUB)])


def _slot_static(j, b, bufs, g_hbm, srcb, effb, acc_sh):
    rows_d, _, ssem_d = bufs[(b + 1) % 3]
    pltpu.make_async_copy(rows_d, acc_sh.at[effb.at[j - 2]], ssem_d).wait()
    rows_g, gsem_g, _ = bufs[(b + 1) % 3]
    pltpu.async_copy(g_hbm.at[srcb.at[j + 1]], rows_g, gsem_g)
    rows_v, gsem, ssem = bufs[b]
    pltpu.make_async_copy(g_hbm.at[srcb.at[j]], rows_v, gsem).wait()
    pltpu.async_copy(rows_v, acc_sh.at[effb.at[j]], ssem, add=True)


@jax.jit
def _msg_pass(g, edge2, eff2):
    f = pl.kernel(
        _msg_body,
        out_type=jax.ShapeDtypeStruct((NC, N_PAD, D), jnp.float32),
        mesh=_mesh,
        compiler_params=_sc_params,
        scratch_types=[
            pltpu.VMEM((N_CHUNK, MSG_K), jnp.int32),
            pltpu.VMEM((N_CHUNK, MSG_K), jnp.int32),
            pltpu.VMEM((MSG_K, D), jnp.float32),
            pltpu.VMEM((MSG_K, D), jnp.float32),
            pltpu.VMEM((MSG_K, D), jnp.float32),
            pltpu.VMEM_SHARED((N_PAD, D), jnp.float32),
            pltpu.SemaphoreType.DMA,
            pltpu.SemaphoreType.DMA,
            pltpu.SemaphoreType.DMA,
            pltpu.SemaphoreType.DMA,
            pltpu.SemaphoreType.DMA,
            pltpu.SemaphoreType.DMA,
        ],
    )
    return f(g, edge2, eff2)


def _dis_from_degp(degp_ref):
    deg = 1.0 + degp_ref[0, :, 0:1] + degp_ref[1, :, 0:1]
    return lax.rsqrt(deg)


def _stage_h_body(x_ref, w1_ref, h1_ref):
    h1_ref[...] = jnp.dot(x_ref[...], w1_ref[...],
                          preferred_element_type=jnp.float32)


def _stage_scale_body(h1_ref, degp_ref, g1_ref):
    g1_ref[...] = h1_ref[...] * _dis_from_degp(degp_ref)


def _stage_b_body(accp_ref, g1_ref, degp_ref, w2_ref, b1_ref, g2_ref):
    dis = _dis_from_degp(degp_ref)
    a = accp_ref[0] + accp_ref[1]
    out1 = jnp.maximum((a + g1_ref[...]) * dis + b1_ref[...], 0.0)
    h2 = jnp.dot(out1, w2_ref[...], preferred_element_type=jnp.float32)
    g2_ref[...] = h2 * dis


def _stage_c_body(accp_ref, g2_ref, degp_ref, b2_ref, out_ref):
    dis = _dis_from_degp(degp_ref)
    a = accp_ref[0] + accp_ref[1]
    out_ref[...] = (a + g2_ref[...]) * dis + b2_ref[...]


_TC_R = 1000

_g_spec = pl.BlockSpec((_TC_R, D), lambda i: (i, 0))
_degp_spec = pl.BlockSpec((NC, _TC_R, 16), lambda i: (0, i, 0))
_accp_spec = pl.BlockSpec((NC, _TC_R, D), lambda i: (0, i, 0))
_w_spec = pl.BlockSpec((D, D), lambda i: (0, 0))
_b_spec = pl.BlockSpec((1, D), lambda i: (0, 0))
_grid = (N_NODES // _TC_R,)
_g_shape = jax.ShapeDtypeStruct((N_NODES, D), jnp.float32)


@jax.jit
def _stage_h(x, w1):
    return pl.pallas_call(
        _stage_h_body,
        grid=_grid,
        in_specs=[_g_spec, _w_spec],
        out_specs=_g_spec,
        out_shape=_g_shape,
    )(x, w1)


@jax.jit
def _stage_scale(h1, degp):
    return pl.pallas_call(
        _stage_scale_body,
        grid=_grid,
        in_specs=[_g_spec, _degp_spec],
        out_specs=_g_spec,
        out_shape=_g_shape,
    )(h1, degp)


@jax.jit
def _stage_b(accp, g1, degp, w2, b1):
    return pl.pallas_call(
        _stage_b_body,
        grid=_grid,
        in_specs=[_accp_spec, _g_spec, _degp_spec, _w_spec, _b_spec],
        out_specs=_g_spec,
        out_shape=_g_shape,
    )(accp, g1, degp, w2, b1)


@jax.jit
def _stage_c(accp, g2, degp, b2):
    return pl.pallas_call(
        _stage_c_body,
        grid=_grid,
        in_specs=[_accp_spec, _g_spec, _degp_spec, _b_spec],
        out_specs=_g_spec,
        out_shape=_g_shape,
    )(accp, g2, degp, b2)


def kernel(x, edge_index, W1, b1, W2, b2):
    edge2 = edge_index.astype(jnp.int32).reshape(2, E_EDGES // MSG_K, MSG_K)
    degp, eff2 = _deg_pass(edge2)
    h1 = _stage_h(x, W1)
    g1 = _stage_scale(h1, degp)
    acc1 = _msg_pass(g1, edge2, eff2)
    g2 = _stage_b(acc1, g1, degp, W2, b1.reshape(1, D))
    acc2 = _msg_pass(g2, edge2, eff2)
    return _stage_c(acc2, g2, degp, b2.reshape(1, D))


# speedup vs baseline: 38.1976x; 1.0212x over previous
import jax
import jax.numpy as jnp
from jax import lax
from jax.experimental import pallas as pl
from jax.experimental.pallas import tpu as pltpu
from jax.experimental.pallas import tpu_sc as plsc

N_NODES = 10000
E_EDGES = 320000
D = 128
NC, NS = 2, 16
NW = NC * NS
E_PER_W = E_EDGES // NW
N_PAD = 10016
ROWS_PER_SUB = N_PAD // NS
MSG_K = 80

_mesh = plsc.VectorSubcoreMesh(core_axis_name="c", subcore_axis_name="s")
_sc_params = pltpu.CompilerParams(use_tc_tiling_on_sc=False)


def _zero_vmem(zrows, nrows, width):
    zv = jnp.zeros((16,), jnp.float32)

    @pl.loop(0, nrows)
    def _(i):
        @pl.loop(0, width, step=16)
        def _(j):
            zrows[i, pl.ds(j, 16)] = zv


def _zero_shared(zrows, znrows, acc_sh, sub, zsem):
    base = sub * ROWS_PER_SUB
    nfull = ROWS_PER_SUB // znrows
    rem = ROWS_PER_SUB % znrows

    @pl.loop(0, nfull)
    def _(j):
        pltpu.async_copy(zrows, acc_sh.at[pl.ds(base + j * znrows, znrows)],
                         zsem)

    if rem:
        pltpu.async_copy(zrows.at[pl.ds(0, rem)],
                         acc_sh.at[pl.ds(base + nfull * znrows, rem)], zsem)

    @pl.loop(0, nfull)
    def _(j):
        pltpu.make_async_copy(
            zrows, acc_sh.at[pl.ds(base + j * znrows, znrows)], zsem).wait()

    if rem:
        pltpu.make_async_copy(
            zrows.at[pl.ds(0, rem)],
            acc_sh.at[pl.ds(base + nfull * znrows, rem)], zsem).wait()


N_CHUNK = E_PER_W // MSG_K


def _deg_body(edge2_hbm, degp_hbm, eff2_hbm, srcb, effb, ones_v, zrows,
              acc_sh, sem, zsem, esem):
    c = lax.axis_index("c")
    s = lax.axis_index("s")
    row0 = (c * NS + s) * N_CHUNK

    pltpu.sync_copy(edge2_hbm.at[0].at[pl.ds(row0, N_CHUNK)], srcb)
    pltpu.sync_copy(edge2_hbm.at[1].at[pl.ds(row0, N_CHUNK)], effb)

    @pl.loop(0, N_CHUNK)
    def _(j):
        @pl.loop(0, MSG_K, step=16)
        def _(i):
            sv = srcb[j, pl.ds(i, 16)]
            dv = effb[j, pl.ds(i, 16)]
            effb[j, pl.ds(i, 16)] = jnp.where(sv == dv, N_NODES, dv)

    pltpu.async_copy(effb, eff2_hbm.at[pl.ds(row0, N_CHUNK)], esem)

    lanes = lax.iota(jnp.int32, 16)
    one_row = jnp.where(lanes == 0, 1.0, 0.0).astype(jnp.float32)

    @pl.loop(0, MSG_K)
    def _(i):
        ones_v[i, :] = one_row

    _zero_vmem(zrows, ZDEG, 16)
    _zero_shared(zrows, ZDEG, acc_sh, s, zsem)
    plsc.subcore_barrier()

    @pl.loop(0, N_CHUNK)
    def _(j):
        pltpu.async_copy(ones_v, acc_sh.at[effb.at[j]], sem, add=True)

    @pl.loop(0, N_CHUNK)
    def _(j):
        pltpu.make_async_copy(ones_v, acc_sh.at[effb.at[j]], sem).wait()

    pltpu.make_async_copy(effb, eff2_hbm.at[pl.ds(row0, N_CHUNK)], esem).wait()

    plsc.subcore_barrier()
    pltpu.sync_copy(acc_sh.at[pl.ds(s * ROWS_PER_SUB, ROWS_PER_SUB)],
                    degp_hbm.at[c].at[pl.ds(s * ROWS_PER_SUB, ROWS_PER_SUB)])


ZDEG = 128


@jax.jit
def _deg_pass(edge2):
    f = pl.kernel(
        _deg_body,
        out_type=(jax.ShapeDtypeStruct((NC, N_PAD, 16), jnp.float32),
                  jax.ShapeDtypeStruct((E_EDGES // MSG_K, MSG_K), jnp.int32)),
        mesh=_mesh,
        compiler_params=_sc_params,
        scratch_types=[
            pltpu.VMEM((N_CHUNK, MSG_K), jnp.int32),
            pltpu.VMEM((N_CHUNK, MSG_K), jnp.int32),
            pltpu.VMEM((MSG_K, 16), jnp.float32),
            pltpu.VMEM((ZDEG, 16), jnp.float32),
            pltpu.VMEM_SHARED((N_PAD, 16), jnp.float32),
            pltpu.SemaphoreType.DMA,
            pltpu.SemaphoreType.DMA,
            pltpu.SemaphoreType.DMA,
        ],
    )
    return f(edge2)


def _msg_body(g_hbm, edge2_hbm, eff2_hbm, accp_hbm,
              srcb, effb, rows_a, rows_b, rows_c, acc_sh,
              gsem_a, gsem_b, gsem_c, ssem_a, ssem_b, ssem_c):
    c = lax.axis_index("c")
    s = lax.axis_index("s")
    row0 = (c * NS + s) * N_CHUNK

    pltpu.sync_copy(edge2_hbm.at[0].at[pl.ds(row0, N_CHUNK)], srcb)
    pltpu.sync_copy(eff2_hbm.at[pl.ds(row0, N_CHUNK)], effb)
    _zero_vmem(rows_a, MSG_K, D)
    _zero_shared(rows_a, MSG_K, acc_sh, s, gsem_a)
    plsc.subcore_barrier()

    bufs = ((rows_a, gsem_a, ssem_a),
            (rows_b, gsem_b, ssem_b),
            (rows_c, gsem_c, ssem_c))

    def issue(j, b):
        rows_v, gsem, _ = bufs[b]
        pltpu.async_copy(g_hbm.at[srcb.at[j]], rows_v, gsem)

    def proc(j, b):
        rows_v, gsem, ssem = bufs[b]
        pltpu.make_async_copy(g_hbm.at[srcb.at[j]], rows_v, gsem).wait()
        pltpu.async_copy(rows_v, acc_sh.at[effb.at[j]], ssem, add=True)

    def drain(j, b):
        rows_v, _, ssem = bufs[b]
        pltpu.make_async_copy(rows_v, acc_sh.at[effb.at[j]], ssem).wait()

    def slot(j, drain_behind=True, gather_ahead=True):
        if drain_behind:
            drain(j - 2, (j + 1) % 3)
        if gather_ahead:
            issue(j + 1, (j + 1) % 3)
        proc(j, j % 3)

    issue(0, 0)
    slot(0, drain_behind=False)
    slot(1, drain_behind=False)

    @pl.loop(2, N_CHUNK - 3, step=3)
    def _(j):
        jj = j
        for k in range(3):
            b1 = (2 + k) % 3
            _slot_static(jj + k, b1, bufs, g_hbm, srcb, effb, acc_sh)

    slot(N_CHUNK - 3)
    slot(N_CHUNK - 2)
    slot(N_CHUNK - 1, gather_ahead=False)
    drain(N_CHUNK - 2, (N_CHUNK - 2) % 3)
    drain(N_CHUNK - 1, (N_CHUNK - 1) % 3)

    plsc.subcore_barrier()
    pltpu.sync_copy(acc_sh.at[pl.ds(s * ROWS_PER_SUB, ROWS_PER_SUB)],
                    accp_hbm.at[c].at[pl.ds(s * ROWS_PER_SUB, ROWS_PER_SUB)])


def _slot_static(j, b, bufs, g_hbm, srcb, effb, acc_sh):
    rows_d, _, ssem_d = bufs[(b + 1) % 3]
    pltpu.make_async_copy(rows_d, acc_sh.at[effb.at[j - 2]], ssem_d).wait()
    rows_g, gsem_g, _ = bufs[(b + 1) % 3]
    pltpu.async_copy(g_hbm.at[srcb.at[j + 1]], rows_g, gsem_g)
    rows_v, gsem, ssem = bufs[b]
    pltpu.make_async_copy(g_hbm.at[srcb.at[j]], rows_v, gsem).wait()
    pltpu.async_copy(rows_v, acc_sh.at[effb.at[j]], ssem, add=True)


@jax.jit
def _msg_pass(g, edge2, eff2):
    f = pl.kernel(
        _msg_body,
        out_type=jax.ShapeDtypeStruct((NC, N_PAD, D), jnp.float32),
        mesh=_mesh,
        compiler_params=_sc_params,
        scratch_types=[
            pltpu.VMEM((N_CHUNK, MSG_K), jnp.int32),
            pltpu.VMEM((N_CHUNK, MSG_K), jnp.int32),
            pltpu.VMEM((MSG_K, D), jnp.float32),
            pltpu.VMEM((MSG_K, D), jnp.float32),
            pltpu.VMEM((MSG_K, D), jnp.float32),
            pltpu.VMEM_SHARED((N_PAD, D), jnp.float32),
            pltpu.SemaphoreType.DMA,
            pltpu.SemaphoreType.DMA,
            pltpu.SemaphoreType.DMA,
            pltpu.SemaphoreType.DMA,
            pltpu.SemaphoreType.DMA,
            pltpu.SemaphoreType.DMA,
        ],
    )
    return f(g, edge2, eff2)


def _dis_from_degp(degp_ref):
    deg = 1.0 + degp_ref[0, :, 0:1] + degp_ref[1, :, 0:1]
    return lax.rsqrt(deg)


def _stage_h_body(x_ref, w1_ref, h1_ref):
    h1_ref[...] = jnp.dot(x_ref[...], w1_ref[...],
                          preferred_element_type=jnp.float32)


def _stage_scale_body(h1_ref, degp_ref, g1_ref):
    g1_ref[...] = h1_ref[...] * _dis_from_degp(degp_ref)


def _stage_b_body(accp_ref, g1_ref, degp_ref, w2_ref, b1_ref, g2_ref):
    dis = _dis_from_degp(degp_ref)
    a = accp_ref[0] + accp_ref[1]
    out1 = jnp.maximum((a + g1_ref[...]) * dis + b1_ref[...], 0.0)
    h2 = jnp.dot(out1, w2_ref[...], preferred_element_type=jnp.float32)
    g2_ref[...] = h2 * dis


def _stage_c_body(accp_ref, g2_ref, degp_ref, b2_ref, out_ref):
    dis = _dis_from_degp(degp_ref)
    a = accp_ref[0] + accp_ref[1]
    out_ref[...] = (a + g2_ref[...]) * dis + b2_ref[...]


_TC_R = 2000

_g_spec = pl.BlockSpec((_TC_R, D), lambda i: (i, 0))
_degp_spec = pl.BlockSpec((NC, _TC_R, 16), lambda i: (0, i, 0))
_accp_spec = pl.BlockSpec((NC, _TC_R, D), lambda i: (0, i, 0))
_w_spec = pl.BlockSpec((D, D), lambda i: (0, 0))
_b_spec = pl.BlockSpec((1, D), lambda i: (0, 0))
_grid = (N_NODES // _TC_R,)
_g_shape = jax.ShapeDtypeStruct((N_NODES, D), jnp.float32)


@jax.jit
def _stage_h(x, w1):
    return pl.pallas_call(
        _stage_h_body,
        grid=_grid,
        in_specs=[_g_spec, _w_spec],
        out_specs=_g_spec,
        out_shape=_g_shape,
    )(x, w1)


@jax.jit
def _stage_scale(h1, degp):
    return pl.pallas_call(
        _stage_scale_body,
        grid=_grid,
        in_specs=[_g_spec, _degp_spec],
        out_specs=_g_spec,
        out_shape=_g_shape,
    )(h1, degp)


@jax.jit
def _stage_b(accp, g1, degp, w2, b1):
    return pl.pallas_call(
        _stage_b_body,
        grid=_grid,
        in_specs=[_accp_spec, _g_spec, _degp_spec, _w_spec, _b_spec],
        out_specs=_g_spec,
        out_shape=_g_shape,
    )(accp, g1, degp, w2, b1)


@jax.jit
def _stage_c(accp, g2, degp, b2):
    return pl.pallas_call(
        _stage_c_body,
        grid=_grid,
        in_specs=[_accp_spec, _g_spec, _degp_spec, _b_spec],
        out_specs=_g_spec,
        out_shape=_g_shape,
    )(accp, g2, degp, b2)


def kernel(x, edge_index, W1, b1, W2, b2):
    edge2 = edge_index.astype(jnp.int32).reshape(2, E_EDGES // MSG_K, MSG_K)
    degp, eff2 = _deg_pass(edge2)
    h1 = _stage_h(x, W1)
    g1 = _stage_scale(h1, degp)
    acc1 = _msg_pass(g1, edge2, eff2)
    g2 = _stage_b(acc1, g1, degp, W2, b1.reshape(1, D))
    acc2 = _msg_pass(g2, edge2, eff2)
    return _stage_c(acc2, g2, degp, b2.reshape(1, D))
